# baseline (speedup 1.0000x reference)
import functools

import jax
import jax.numpy as jnp
from jax import lax
from jax.experimental import pallas as pl
from jax.experimental.pallas import tpu as pltpu
from jax.experimental.pallas import tpu_sc as plsc

B = 32
C = 384
H = 56
W = 56
D = H * W
CH = 16
NCHUNK = C // CH
NPAIR = NCHUNK // 2


def _permute_rows(in2d, idx_i32):
    mesh = plsc.VectorSubcoreMesh(core_axis_name="c", subcore_axis_name="s")
    num_cores = mesh.num_cores

    @functools.partial(
        pl.kernel,
        out_type=jax.ShapeDtypeStruct((B * C, D), jnp.float32),
        mesh=mesh,
        compiler_params=pltpu.CompilerParams(use_tc_tiling_on_sc=False),
        scratch_types=[
            pltpu.VMEM((C,), jnp.int32),
            pltpu.VMEM((2, CH, D), jnp.float32),
            pltpu.SemaphoreType.DMA,
        ],
    )
    def k(in_hbm, idx_hbm, out_hbm, idx_v, buf, sem):
        wid = lax.axis_index("s") * num_cores + lax.axis_index("c")
        base = wid * C
        pltpu.sync_copy(idx_hbm, idx_v)

        def gather(i, slot):
            rows = idx_v[pl.ds(i * CH, CH)] + base
            pltpu.async_copy(in_hbm.at[rows], buf.at[slot], sem)

        def wait_one(slot):
            pltpu.make_async_copy(in_hbm.at[pl.ds(0, CH)], buf.at[slot], sem).wait()

        def put(i, slot):
            pltpu.sync_copy(buf.at[slot], out_hbm.at[pl.ds(base + i * CH, CH)])

        gather(0, 0)

        def body(p, _):
            i0 = p * 2
            gather(i0 + 1, 1)
            wait_one(0)
            put(i0, 0)

            @pl.when(i0 + 2 < NCHUNK)
            def _():
                gather(i0 + 2, 0)

            wait_one(1)
            put(i0 + 1, 1)
            return 0

        lax.fori_loop(0, NPAIR, body, 0)

    return k(in2d, idx_i32)


def kernel(input, indices, indices_inverse):
    in2d = input.reshape(B * C, D)
    idx = indices.astype(jnp.int32)
    out2d = _permute_rows(in2d, idx)
    return out2d.reshape(B, C, H, W)
